# baseline (speedup 1.0000x reference)
import jax
import jax.numpy as jnp
from jax import lax
from jax.experimental import pallas as pl
from jax.experimental.pallas import tpu as pltpu
from jax.experimental.pallas import tpu_sc as plsc

NUM_HEADS = 16
NUM_BUCKETS = 32
Q_LEN = 2048
K_LEN = 2048

NC, NS, L = 2, 16, 16
NW = NC * NS
ROWS_PER_W = NUM_HEADS * Q_LEN // NW
CHUNK = 16
NCHUNK = ROWS_PER_W // CHUNK
NSHIFT = 8
SW = 4096
BAND_LO = 1920
BAND_VECS = 15

_THRESH = (21, 26, 32, 40, 48, 57, 68, 80, 94, 110)


def _sc_body(tabT_hbm, out_hbm, col_v, strip_v, sem):
    wid = lax.axis_index("c") * NS + lax.axis_index("s")
    h = wid // 2
    half = wid % 2
    pltpu.sync_copy(tabT_hbm.at[h], col_v)
    c31 = plsc.load_gather(col_v, [jnp.full((L,), 31, jnp.int32)])

    def fill(t, carry):
        for r in range(NSHIFT):
            strip_v[pl.ds(r * SW + t * L, L)] = c31
        return carry

    lax.fori_loop(0, SW // L, fill, 0)

    lane = lax.broadcasted_iota(jnp.int32, (L,), 0)

    def band(v, carry):
        j0 = BAND_LO + v * L
        j = lane + j0
        for r in range(NSHIFT):
            a = jnp.abs(j + (r - 2047))
            cnt = jnp.full((L,), 21, jnp.int32)
            for t in _THRESH:
                cnt = cnt + (a >= t).astype(jnp.int32)
            b = jnp.where(a < 16, a, cnt)
            strip_v[pl.ds(r * SW + j0, L)] = plsc.load_gather(col_v, [b])
        return carry

    lax.fori_loop(0, BAND_VECS, band, 0)

    def src_off(Q, i):
        if i < 8:
            return (7 - i) * SW + 2040 - Q
        return (15 - i) * SW + 2032 - Q

    row0 = h * Q_LEN + half * (Q_LEN // 2)

    def chunk(g, carry):
        Q = half * (Q_LEN // 2) + g * CHUNK
        for i in range(CHUNK):
            pltpu.async_copy(
                strip_v.at[pl.ds(src_off(Q, i), K_LEN)],
                out_hbm.at[row0 + g * CHUNK + i],
                sem,
            )

        @pl.when(g > 0)
        def _():
            Qp = half * (Q_LEN // 2) + (g - 1) * CHUNK
            for i in range(CHUNK):
                pltpu.make_async_copy(
                    strip_v.at[pl.ds(src_off(Qp, i), K_LEN)],
                    out_hbm.at[row0 + (g - 1) * CHUNK + i],
                    sem,
                ).wait()

        return carry

    lax.fori_loop(0, NCHUNK, chunk, 0)

    Qp = half * (Q_LEN // 2) + (NCHUNK - 1) * CHUNK
    for i in range(CHUNK):
        pltpu.make_async_copy(
            strip_v.at[pl.ds(src_off(Qp, i), K_LEN)],
            out_hbm.at[row0 + (NCHUNK - 1) * CHUNK + i],
            sem,
        ).wait()


def kernel(query_length, key_length, bias_table):
    del query_length, key_length
    tabT = jnp.transpose(bias_table, (1, 0))
    f = pl.kernel(
        _sc_body,
        out_type=jax.ShapeDtypeStruct((NUM_HEADS * Q_LEN, K_LEN), jnp.float32),
        mesh=plsc.VectorSubcoreMesh(core_axis_name="c", subcore_axis_name="s"),
        scratch_types=[
            pltpu.VMEM((NUM_BUCKETS,), jnp.float32),
            pltpu.VMEM((NSHIFT * SW,), jnp.float32),
            pltpu.SemaphoreType.DMA,
        ],
        compiler_params=pltpu.CompilerParams(
            needs_layout_passes=False, use_tc_tiling_on_sc=False
        ),
    )
    out = f(tabT)
    return out.reshape(1, NUM_HEADS, Q_LEN, K_LEN)


# speedup vs baseline: 3.1635x; 3.1635x over previous
import jax
import jax.numpy as jnp
from jax import lax
from jax.experimental import pallas as pl
from jax.experimental.pallas import tpu as pltpu
from jax.experimental.pallas import tpu_sc as plsc

NUM_HEADS = 16
NUM_BUCKETS = 32
Q_LEN = 2048
K_LEN = 2048

NC, NS, L = 2, 16, 16
ROWS_PER_W = 1024
C_MIN, C_MAX = -112, 232
NSLOT = (C_MAX - C_MIN) // 8 + 1
M_W = 512
CBC_ROWS = 256

_THRESH = (21, 26, 32, 40, 48, 57, 68, 80, 94, 110)


def _const_runs(n):
    runs, sz = [], CBC_ROWS
    while n > 0:
        while sz > n:
            sz //= 2
        runs.append(sz)
        n -= sz
    return runs


def _sc_body(tabT_hbm, out_hbm, col_v, m_v, tl_v, cbc_v, sem):
    wid = lax.axis_index("c") * NS + lax.axis_index("s")
    h = wid // 2
    half = wid % 2
    pltpu.sync_copy(tabT_hbm.at[h], col_v)
    c31 = plsc.load_gather(col_v, [jnp.full((L,), 31, jnp.int32)])

    lane = lax.broadcasted_iota(jnp.int32, (L,), 0)

    def mbuild(v, carry):
        j = lane + v * L
        a = jnp.abs(j - 256)
        cnt = jnp.full((L,), 21, jnp.int32)
        for t in _THRESH:
            cnt = cnt + (a >= t).astype(jnp.int32)
        b = jnp.where(a < 16, a, cnt)
        m_v[pl.ds(v * L, L)] = plsc.load_gather(col_v, [b])
        return carry

    lax.fori_loop(0, M_W // L, mbuild, 0)

    def tlbuild(s, carry):
        for i in range(8):
            off = 368 - 8 * s - i
            for j in range(128 // L):
                tl_v[8 * s + i, pl.ds(j * L, L)] = m_v[pl.ds(off + j * L, L)]
        return carry

    lax.fori_loop(0, NSLOT, tlbuild, 0)

    def cbuild(r, carry):
        for j in range(128 // L):
            cbc_v[r, pl.ds(j * L, L)] = c31
        return carry

    lax.fori_loop(0, CBC_ROWS, cbuild, 0)

    row_off = h * Q_LEN

    for half_s in (0, 1):

        @pl.when(half == half_s)
        def _(half_s=half_s):
            r0 = half_s * ROWS_PER_W
            descs = []
            for t in range(K_LEN // 128):
                lo8 = max(128 * t + C_MIN, r0)
                hi8 = min(128 * t + C_MAX, r0 + ROWS_PER_W - 8)
                segs = []
                if lo8 <= hi8:
                    if lo8 > r0:
                        segs.append((r0, lo8 - r0, False))
                    segs.append((lo8, hi8 + 8 - lo8, True))
                    if hi8 + 8 < r0 + ROWS_PER_W:
                        segs.append((hi8 + 8, r0 + ROWS_PER_W - hi8 - 8, False))
                else:
                    segs.append((r0, ROWS_PER_W, False))
                for start, n, is_band in segs:
                    if is_band:
                        slot_lo = (start - 128 * t - C_MIN) // 8
                        descs.append(
                            pltpu.async_copy(
                                tl_v.at[pl.ds(8 * slot_lo, n), :],
                                out_hbm.at[
                                    pl.ds(row_off + start, n),
                                    pl.ds(128 * t, 128),
                                ],
                                sem,
                            )
                        )
                    else:
                        pos = start
                        for sz in _const_runs(n):
                            descs.append(
                                pltpu.async_copy(
                                    cbc_v.at[pl.ds(0, sz), :],
                                    out_hbm.at[
                                        pl.ds(row_off + pos, sz),
                                        pl.ds(128 * t, 128),
                                    ],
                                    sem,
                                )
                            )
                            pos += sz
            for d in descs:
                d.wait()


def kernel(query_length, key_length, bias_table):
    del query_length, key_length
    tabT = jnp.transpose(bias_table, (1, 0))
    f = pl.kernel(
        _sc_body,
        out_type=jax.ShapeDtypeStruct((NUM_HEADS * Q_LEN, K_LEN), jnp.float32),
        mesh=plsc.VectorSubcoreMesh(core_axis_name="c", subcore_axis_name="s"),
        scratch_types=[
            pltpu.VMEM((NUM_BUCKETS,), jnp.float32),
            pltpu.VMEM((M_W,), jnp.float32),
            pltpu.VMEM((8 * NSLOT, 128), jnp.float32),
            pltpu.VMEM((CBC_ROWS, 128), jnp.float32),
            pltpu.SemaphoreType.DMA,
        ],
        compiler_params=pltpu.CompilerParams(needs_layout_passes=False),
    )
    out = f(tabT)
    return out.reshape(1, NUM_HEADS, Q_LEN, K_LEN)


# speedup vs baseline: 3.3340x; 1.0539x over previous
import jax
import jax.numpy as jnp
from jax import lax
from jax.experimental import pallas as pl
from jax.experimental.pallas import tpu as pltpu
from jax.experimental.pallas import tpu_sc as plsc

NUM_HEADS = 16
NUM_BUCKETS = 32
Q_LEN = 2048
K_LEN = 2048

NC, NS, L = 2, 16, 16
ROWS_PER_W = 1024
C_MIN, C_MAX = -112, 232
NSLOT = (C_MAX - C_MIN) // 8 + 1
M_W = 512
CBC_ROWS = 64

_THRESH = (21, 26, 32, 40, 48, 57, 68, 80, 94, 110)


def _const_runs(n):
    runs, sz = [], CBC_ROWS
    while n > 0:
        while sz > n:
            sz //= 2
        runs.append(sz)
        n -= sz
    return runs


def _sc_body(tabT_hbm, out_hbm, col_v, m_v, tl_v, cbc_v, sem):
    wid = lax.axis_index("c") * NS + lax.axis_index("s")
    h = wid // 2
    half = wid % 2
    pltpu.sync_copy(tabT_hbm.at[h], col_v)
    c31 = plsc.load_gather(col_v, [jnp.full((L,), 31, jnp.int32)])

    lane = lax.broadcasted_iota(jnp.int32, (L,), 0)

    def cbuild(r, carry):
        for j in range(128 // L):
            cbc_v[r, pl.ds(j * L, L)] = c31
        return carry

    lax.fori_loop(0, CBC_ROWS, cbuild, 0)

    row_off = h * Q_LEN

    def _segs(r0):
        out = []
        for t in range(K_LEN // 128):
            lo8 = max(128 * t + C_MIN, r0)
            hi8 = min(128 * t + C_MAX, r0 + ROWS_PER_W - 8)
            segs = []
            if lo8 <= hi8:
                if lo8 > r0:
                    segs.append((r0, lo8 - r0, False))
                segs.append((lo8, hi8 + 8 - lo8, True))
                if hi8 + 8 < r0 + ROWS_PER_W:
                    segs.append((hi8 + 8, r0 + ROWS_PER_W - hi8 - 8, False))
            else:
                segs.append((r0, ROWS_PER_W, False))
            out.append((t, segs))
        return out

    for half_s in (0, 1):

        @pl.when(half == half_s)
        def _(half_s=half_s):
            plan = _segs(half_s * ROWS_PER_W)
            descs = []
            for t, segs in plan:
                for start, n, is_band in segs:
                    if is_band:
                        continue
                    pos = start
                    for sz in _const_runs(n):
                        descs.append(
                            pltpu.async_copy(
                                cbc_v.at[pl.ds(0, sz), :],
                                out_hbm.at[
                                    pl.ds(row_off + pos, sz),
                                    pl.ds(128 * t, 128),
                                ],
                                sem,
                            )
                        )
                        pos += sz

            def mbuild(v, carry):
                j = lane + v * L
                a = jnp.abs(j - 256)
                cnt = jnp.full((L,), 21, jnp.int32)
                for tt in _THRESH:
                    cnt = cnt + (a >= tt).astype(jnp.int32)
                b = jnp.where(a < 16, a, cnt)
                m_v[pl.ds(v * L, L)] = plsc.load_gather(col_v, [b])
                return carry

            lax.fori_loop(0, M_W // L, mbuild, 0)

            def tlbuild(s, carry):
                for i in range(8):
                    off = 368 - 8 * s - i
                    for j in range(128 // L):
                        tl_v[8 * s + i, pl.ds(j * L, L)] = m_v[
                            pl.ds(off + j * L, L)
                        ]
                return carry

            lax.fori_loop(0, NSLOT, tlbuild, 0)

            for t, segs in plan:
                for start, n, is_band in segs:
                    if not is_band:
                        continue
                    slot_lo = (start - 128 * t - C_MIN) // 8
                    descs.append(
                        pltpu.async_copy(
                            tl_v.at[pl.ds(8 * slot_lo, n), :],
                            out_hbm.at[
                                pl.ds(row_off + start, n),
                                pl.ds(128 * t, 128),
                            ],
                            sem,
                        )
                    )
            for d in descs:
                d.wait()


def kernel(query_length, key_length, bias_table):
    del query_length, key_length
    tabT = jnp.transpose(bias_table, (1, 0))
    f = pl.kernel(
        _sc_body,
        out_type=jax.ShapeDtypeStruct((NUM_HEADS * Q_LEN, K_LEN), jnp.float32),
        mesh=plsc.VectorSubcoreMesh(core_axis_name="c", subcore_axis_name="s"),
        scratch_types=[
            pltpu.VMEM((NUM_BUCKETS,), jnp.float32),
            pltpu.VMEM((M_W,), jnp.float32),
            pltpu.VMEM((8 * NSLOT, 128), jnp.float32),
            pltpu.VMEM((CBC_ROWS, 128), jnp.float32),
            pltpu.SemaphoreType.DMA,
        ],
        compiler_params=pltpu.CompilerParams(needs_layout_passes=False),
    )
    out = f(tabT)
    return out.reshape(1, NUM_HEADS, Q_LEN, K_LEN)


# speedup vs baseline: 3.4063x; 1.0217x over previous
import jax
import jax.numpy as jnp
from jax import lax
from jax.experimental import pallas as pl
from jax.experimental.pallas import tpu as pltpu
from jax.experimental.pallas import tpu_sc as plsc

NUM_HEADS = 16
NUM_BUCKETS = 32
Q_LEN = 2048
K_LEN = 2048

NC, NS, L = 2, 16, 16
ROWS_PER_W = 1024
C_MIN, C_MAX = -112, 232
NSLOT = (C_MAX - C_MIN) // 8 + 1
M_W = 512
CBC_ROWS = 128

_THRESH = (21, 26, 32, 40, 48, 57, 68, 80, 94, 110)


def _const_runs(n):
    runs, sz = [], CBC_ROWS
    while n > 0:
        while sz > n:
            sz //= 2
        runs.append(sz)
        n -= sz
    return runs


def _sc_body(tabT_hbm, out_hbm, col_v, m_v, tl_v, cbc_v, sem):
    wid = lax.axis_index("c") * NS + lax.axis_index("s")
    h = wid // 2
    half = wid % 2
    pltpu.sync_copy(tabT_hbm.at[h], col_v)
    c31 = plsc.load_gather(col_v, [jnp.full((L,), 31, jnp.int32)])

    lane = lax.broadcasted_iota(jnp.int32, (L,), 0)

    def cbuild(r, carry):
        for j in range(128 // L):
            cbc_v[r, pl.ds(j * L, L)] = c31
        return carry

    lax.fori_loop(0, CBC_ROWS, cbuild, 0)

    row_off = h * Q_LEN

    def _segs(r0):
        out = []
        for t in range(K_LEN // 128):
            lo8 = max(128 * t + C_MIN, r0)
            hi8 = min(128 * t + C_MAX, r0 + ROWS_PER_W - 8)
            segs = []
            if lo8 <= hi8:
                if lo8 > r0:
                    segs.append((r0, lo8 - r0, False))
                segs.append((lo8, hi8 + 8 - lo8, True))
                if hi8 + 8 < r0 + ROWS_PER_W:
                    segs.append((hi8 + 8, r0 + ROWS_PER_W - hi8 - 8, False))
            else:
                segs.append((r0, ROWS_PER_W, False))
            out.append((t, segs))
        return out

    for half_s in (0, 1):

        @pl.when(half == half_s)
        def _(half_s=half_s):
            plan = _segs(half_s * ROWS_PER_W)
            descs = []
            for t, segs in plan:
                for start, n, is_band in segs:
                    if is_band:
                        continue
                    pos = start
                    for sz in _const_runs(n):
                        descs.append(
                            pltpu.async_copy(
                                cbc_v.at[pl.ds(0, sz), :],
                                out_hbm.at[
                                    pl.ds(row_off + pos, sz),
                                    pl.ds(128 * t, 128),
                                ],
                                sem,
                            )
                        )
                        pos += sz

            def mbuild(v, carry):
                j = lane + v * L
                a = jnp.abs(j - 256)
                cnt = jnp.full((L,), 21, jnp.int32)
                for tt in _THRESH:
                    cnt = cnt + (a >= tt).astype(jnp.int32)
                b = jnp.where(a < 16, a, cnt)
                m_v[pl.ds(v * L, L)] = plsc.load_gather(col_v, [b])
                return carry

            lax.fori_loop(0, M_W // L, mbuild, 0)

            def tlbuild(s, carry):
                for i in range(8):
                    off = 368 - 8 * s - i
                    for j in range(128 // L):
                        tl_v[8 * s + i, pl.ds(j * L, L)] = m_v[
                            pl.ds(off + j * L, L)
                        ]
                return carry

            lax.fori_loop(0, NSLOT, tlbuild, 0)

            for t, segs in plan:
                for start, n, is_band in segs:
                    if not is_band:
                        continue
                    slot_lo = (start - 128 * t - C_MIN) // 8
                    descs.append(
                        pltpu.async_copy(
                            tl_v.at[pl.ds(8 * slot_lo, n), :],
                            out_hbm.at[
                                pl.ds(row_off + start, n),
                                pl.ds(128 * t, 128),
                            ],
                            sem,
                        )
                    )
            for d in descs:
                d.wait()


def kernel(query_length, key_length, bias_table):
    del query_length, key_length
    tabT = jnp.transpose(bias_table, (1, 0))
    f = pl.kernel(
        _sc_body,
        out_type=jax.ShapeDtypeStruct((NUM_HEADS * Q_LEN, K_LEN), jnp.float32),
        mesh=plsc.VectorSubcoreMesh(core_axis_name="c", subcore_axis_name="s"),
        scratch_types=[
            pltpu.VMEM((NUM_BUCKETS,), jnp.float32),
            pltpu.VMEM((M_W,), jnp.float32),
            pltpu.VMEM((8 * NSLOT, 128), jnp.float32),
            pltpu.VMEM((CBC_ROWS, 128), jnp.float32),
            pltpu.SemaphoreType.DMA,
        ],
        compiler_params=pltpu.CompilerParams(needs_layout_passes=False),
    )
    out = f(tabT)
    return out.reshape(1, NUM_HEADS, Q_LEN, K_LEN)


# speedup vs baseline: 3.4311x; 1.0073x over previous
import jax
import jax.numpy as jnp
from jax import lax
from jax.experimental import pallas as pl
from jax.experimental.pallas import tpu as pltpu
from jax.experimental.pallas import tpu_sc as plsc

NUM_HEADS = 16
NUM_BUCKETS = 32
Q_LEN = 2048
K_LEN = 2048

NC, NS, L = 2, 16, 16
ROWS_PER_W = 1024
C_MIN, C_MAX = -112, 232
NSLOT = (C_MAX - C_MIN) // 8 + 1
M_W = 512
CBC_ROWS = 256

_THRESH = (21, 26, 32, 40, 48, 57, 68, 80, 94, 110)


def _const_runs(n):
    runs, sz = [], CBC_ROWS
    while n > 0:
        while sz > n:
            sz //= 2
        runs.append(sz)
        n -= sz
    return runs


def _sc_body(tabT_hbm, out_hbm, col_v, m_v, tl_v, cbc_v, sem):
    wid = lax.axis_index("c") * NS + lax.axis_index("s")
    h = wid // 2
    half = wid % 2
    pltpu.sync_copy(tabT_hbm.at[h], col_v)
    c31 = plsc.load_gather(col_v, [jnp.full((L,), 31, jnp.int32)])

    lane = lax.broadcasted_iota(jnp.int32, (L,), 0)

    def cbuild(r, carry):
        for j in range(128 // L):
            cbc_v[r, pl.ds(j * L, L)] = c31
        return carry

    lax.fori_loop(0, CBC_ROWS, cbuild, 0)

    row_off = h * Q_LEN

    def _segs(r0):
        out = []
        for t in range(K_LEN // 128):
            lo8 = max(128 * t + C_MIN, r0)
            hi8 = min(128 * t + C_MAX, r0 + ROWS_PER_W - 8)
            segs = []
            if lo8 <= hi8:
                if lo8 > r0:
                    segs.append((r0, lo8 - r0, False))
                segs.append((lo8, hi8 + 8 - lo8, True))
                if hi8 + 8 < r0 + ROWS_PER_W:
                    segs.append((hi8 + 8, r0 + ROWS_PER_W - hi8 - 8, False))
            else:
                segs.append((r0, ROWS_PER_W, False))
            out.append((t, segs))
        return out

    for half_s in (0, 1):

        @pl.when(half == half_s)
        def _(half_s=half_s):
            plan = _segs(half_s * ROWS_PER_W)
            descs = []
            for t, segs in plan:
                for start, n, is_band in segs:
                    if is_band:
                        continue
                    pos = start
                    for sz in _const_runs(n):
                        descs.append(
                            pltpu.async_copy(
                                cbc_v.at[pl.ds(0, sz), :],
                                out_hbm.at[
                                    pl.ds(row_off + pos, sz),
                                    pl.ds(128 * t, 128),
                                ],
                                sem,
                            )
                        )
                        pos += sz

            def mbuild(v, carry):
                j = lane + v * L
                a = jnp.abs(j - 256)
                cnt = jnp.full((L,), 21, jnp.int32)
                for tt in _THRESH:
                    cnt = cnt + (a >= tt).astype(jnp.int32)
                b = jnp.where(a < 16, a, cnt)
                m_v[pl.ds(v * L, L)] = plsc.load_gather(col_v, [b])
                return carry

            lax.fori_loop(0, M_W // L, mbuild, 0)

            def tlbuild(s, carry):
                for i in range(8):
                    off = 368 - 8 * s - i
                    for j in range(128 // L):
                        tl_v[8 * s + i, pl.ds(j * L, L)] = m_v[
                            pl.ds(off + j * L, L)
                        ]
                return carry

            lax.fori_loop(0, NSLOT, tlbuild, 0)

            for t, segs in plan:
                for start, n, is_band in segs:
                    if not is_band:
                        continue
                    slot_lo = (start - 128 * t - C_MIN) // 8
                    descs.append(
                        pltpu.async_copy(
                            tl_v.at[pl.ds(8 * slot_lo, n), :],
                            out_hbm.at[
                                pl.ds(row_off + start, n),
                                pl.ds(128 * t, 128),
                            ],
                            sem,
                        )
                    )
            for d in descs:
                d.wait()


def kernel(query_length, key_length, bias_table):
    del query_length, key_length
    tabT = jnp.transpose(bias_table, (1, 0))
    f = pl.kernel(
        _sc_body,
        out_type=jax.ShapeDtypeStruct((NUM_HEADS * Q_LEN, K_LEN), jnp.float32),
        mesh=plsc.VectorSubcoreMesh(core_axis_name="c", subcore_axis_name="s"),
        scratch_types=[
            pltpu.VMEM((NUM_BUCKETS,), jnp.float32),
            pltpu.VMEM((M_W,), jnp.float32),
            pltpu.VMEM((8 * NSLOT, 128), jnp.float32),
            pltpu.VMEM((CBC_ROWS, 128), jnp.float32),
            pltpu.SemaphoreType.DMA,
        ],
        compiler_params=pltpu.CompilerParams(needs_layout_passes=False),
    )
    out = f(tabT)
    return out.reshape(1, NUM_HEADS, Q_LEN, K_LEN)
